# baseline (speedup 1.0000x reference)
import functools

import jax
import jax.numpy as jnp
from jax import lax
from jax.experimental import pallas as pl
from jax.experimental.pallas import tpu as pltpu
from jax.experimental.pallas import tpu_sc as plsc

N_NODES = 10000
N_EDGES = 160000
NPAD = 10240
NACC = 10016
DUMMY = N_NODES
NTILE = 16
BEDGE = 128
NB = 80
SB = 16
NWIN = NB // SB
EPAD = NTILE * NB * BEDGE
BM = 256


def _acc_rows(s):
  return 9600 if s == 15 else None


def _make_sc_agg(nchk, with_deg):
  mesh = plsc.VectorSubcoreMesh(core_axis_name="c", subcore_axis_name="s")

  out_type = tuple(
      jax.ShapeDtypeStruct((NPAD, 128), jnp.float32)
      for _ in range(nchk + (1 if with_deg else 0)))
  scratch = [
      pltpu.VMEM((SB, BEDGE), jnp.int32),
      pltpu.VMEM((SB, BEDGE), jnp.int32),
      pltpu.VMEM((SB, BEDGE), jnp.int32),
      pltpu.VMEM((SB, BEDGE), jnp.int32),
      pltpu.VMEM((BEDGE, 128), jnp.float32),
      pltpu.VMEM((BEDGE, 128), jnp.float32),
      pltpu.VMEM_SHARED((NACC, 128), jnp.float32),
      pltpu.SemaphoreType.DMA,
      pltpu.SemaphoreType.DMA,
      pltpu.SemaphoreType.DMA,
  ]

  @functools.partial(pl.kernel, mesh=mesh, out_type=out_type,
                     scratch_types=scratch)
  def k(*refs):
    vals = refs[:nchk]
    pos = nchk
    srcp, dstp, zeros_h = refs[pos], refs[pos + 1], refs[pos + 2]
    pos += 3
    if with_deg:
      ones_h = refs[pos]
      pos += 1
    outs = refs[pos:pos + nchk]
    pos += nchk
    if with_deg:
      dout = refs[pos]
      pos += 1
    sw0, sw1, dw0, dw1, rows0, rows1, acc, sem0, sem1, semi = refs[pos:]
    sw = (sw0, sw1)
    dw = (dw0, dw1)

    c = lax.axis_index("c")
    s = lax.axis_index("s")
    row0 = s * 640
    last = s == NTILE - 1

    def zero_acc():
      @pl.when(jnp.logical_not(last))
      def _():
        pltpu.sync_copy(zeros_h, acc.at[pl.ds(row0, 640)])

      @pl.when(last)
      def _():
        pltpu.sync_copy(zeros_h.at[pl.ds(0, 416)],
                        acc.at[pl.ds(9600, 416)])

    def flush(out):
      @pl.when(jnp.logical_not(last))
      def _():
        pltpu.sync_copy(acc.at[pl.ds(row0, 640)], out.at[pl.ds(row0, 640)])

      @pl.when(last)
      def _():
        pltpu.sync_copy(acc.at[pl.ds(9600, 416)],
                        out.at[pl.ds(9600, 416)])

    @pl.when(c == 0)
    def _():
      for ck in range(nchk):
        vck = vals[ck]
        zero_acc()
        plsc.subcore_barrier()

        pltpu.sync_copy(srcp.at[s, pl.ds(0, SB)], sw[0])
        pltpu.sync_copy(dstp.at[s, pl.ds(0, SB)], dw[0])

        for w in range(NWIN):
          cs, cd = sw[w % 2], dw[w % 2]
          if w + 1 < NWIN:
            ns, nd = sw[(w + 1) % 2], dw[(w + 1) % 2]
            off = (w + 1) * SB
            pltpu.async_copy(srcp.at[s, pl.ds(off, SB)], ns, semi)
            pltpu.async_copy(dstp.at[s, pl.ds(off, SB)], nd, semi)

          pltpu.async_copy(vck.at[cs.at[0]], rows0, sem0)

          def pair(i, carry, cs=cs, cd=cd):
            j = 2 * i
            pltpu.make_async_copy(vck.at[cs.at[j]], rows0, sem0).wait()
            pltpu.async_copy(vck.at[cs.at[j + 1]], rows1, sem1)
            pltpu.sync_copy(rows0, acc.at[cd.at[j]], add=True)
            pltpu.make_async_copy(vck.at[cs.at[j + 1]], rows1, sem1).wait()
            pltpu.async_copy(vck.at[cs.at[j + 2]], rows0, sem0)
            pltpu.sync_copy(rows1, acc.at[cd.at[j + 1]], add=True)
            return carry

          lax.fori_loop(0, SB // 2 - 1, pair, 0)
          pltpu.make_async_copy(vck.at[cs.at[SB - 2]], rows0, sem0).wait()
          pltpu.async_copy(vck.at[cs.at[SB - 1]], rows1, sem1)
          pltpu.sync_copy(rows0, acc.at[cd.at[SB - 2]], add=True)
          pltpu.make_async_copy(vck.at[cs.at[SB - 1]], rows1, sem1).wait()
          pltpu.sync_copy(rows1, acc.at[cd.at[SB - 1]], add=True)

          if w + 1 < NWIN:
            off = (w + 1) * SB
            pltpu.make_async_copy(srcp.at[s, pl.ds(off, SB)], ns, semi).wait()
            pltpu.make_async_copy(dstp.at[s, pl.ds(off, SB)], nd, semi).wait()

        plsc.subcore_barrier()
        flush(outs[ck])
        plsc.subcore_barrier()

    if with_deg:
      @pl.when(c == 1)
      def _():
        pltpu.sync_copy(ones_h, rows0)
        zero_acc()
        plsc.subcore_barrier()

        for w in range(NWIN):
          pltpu.sync_copy(dstp.at[s, pl.ds(w * SB, SB)], dw0)

          def dbatch(j, carry):
            pltpu.sync_copy(rows0, acc.at[dw0.at[j]], add=True)
            return carry

          lax.fori_loop(0, SB, dbatch, 0)
        plsc.subcore_barrier()
        flush(dout)

  return k


def _deg_inv(pd_ref):
  return 1.0 / jnp.maximum(pd_ref[:, 0:1], 1.0)


def _make_tc_layer0():
  grid = (NPAD // BM,)

  def body(p_ref, pd_ref, x_ref, wl_ref, bl_ref, wr_ref, o_ref):
    inv = _deg_inv(pd_ref)
    x = jnp.concatenate([x_ref[cc] for cc in range(2)], axis=-1)
    acc = jnp.dot(x, wr_ref[...], preferred_element_type=jnp.float32)
    acc += bl_ref[...]
    agg = jnp.concatenate([p_ref[cc] for cc in range(2)], axis=-1) * inv
    acc += jnp.dot(agg, wl_ref[...], preferred_element_type=jnp.float32)
    h = jnp.maximum(acc, 0.0)
    for co in range(4):
      o_ref[co] = h[:, co * 128:(co + 1) * 128]

  return pl.pallas_call(
      body,
      grid=grid,
      in_specs=[
          pl.BlockSpec((2, BM, 128), lambda i: (0, i, 0)),
          pl.BlockSpec((BM, 128), lambda i: (i, 0)),
          pl.BlockSpec((2, BM, 128), lambda i: (0, i, 0)),
          pl.BlockSpec((256, 512), lambda i: (0, 0)),
          pl.BlockSpec((1, 512), lambda i: (0, 0)),
          pl.BlockSpec((256, 512), lambda i: (0, 0)),
      ],
      out_specs=pl.BlockSpec((4, BM, 128), lambda i: (0, i, 0)),
      out_shape=jax.ShapeDtypeStruct((4, NPAD, 128), jnp.float32),
  )


def _make_tc_layer1():
  grid = (NPAD // BM,)

  def body(p_ref, pd_ref, x_ref, wl_ref, bl_ref, wr_ref, w2_ref,
           z_ref, r_ref):
    inv = _deg_inv(pd_ref)
    x = jnp.concatenate([x_ref[cc] for cc in range(4)], axis=-1)
    acc = jnp.dot(x, wr_ref[...], preferred_element_type=jnp.float32)
    acc += bl_ref[...]
    agg = jnp.concatenate([p_ref[cc] for cc in range(4)], axis=-1) * inv
    acc += jnp.dot(agg, wl_ref[...], preferred_element_type=jnp.float32)
    h = jnp.maximum(acc, 0.0)
    zr = jnp.dot(h, w2_ref[...], preferred_element_type=jnp.float32)
    z_ref[...] = zr[:, :128]
    r_ref[...] = zr[:, 128:]

  return pl.pallas_call(
      body,
      grid=grid,
      in_specs=[
          pl.BlockSpec((4, BM, 128), lambda i: (0, i, 0)),
          pl.BlockSpec((BM, 128), lambda i: (i, 0)),
          pl.BlockSpec((4, BM, 128), lambda i: (0, i, 0)),
          pl.BlockSpec((512, 512), lambda i: (0, 0)),
          pl.BlockSpec((1, 512), lambda i: (0, 0)),
          pl.BlockSpec((512, 512), lambda i: (0, 0)),
          pl.BlockSpec((512, 256), lambda i: (0, 0)),
      ],
      out_specs=[
          pl.BlockSpec((BM, 128), lambda i: (i, 0)),
          pl.BlockSpec((BM, 128), lambda i: (i, 0)),
      ],
      out_shape=[
          jax.ShapeDtypeStruct((NPAD, 128), jnp.float32),
          jax.ShapeDtypeStruct((NPAD, 128), jnp.float32),
      ],
  )


def _make_tc_post2():
  grid = (NPAD // BM,)

  def body(p_ref, pd_ref, r_ref, bl_ref, o_ref):
    inv = _deg_inv(pd_ref)
    o_ref[...] = p_ref[...] * inv + r_ref[...] + bl_ref[...]

  return pl.pallas_call(
      body,
      grid=grid,
      in_specs=[
          pl.BlockSpec((BM, 128), lambda i: (i, 0)),
          pl.BlockSpec((BM, 128), lambda i: (i, 0)),
          pl.BlockSpec((BM, 128), lambda i: (i, 0)),
          pl.BlockSpec((1, 128), lambda i: (0, 0)),
      ],
      out_specs=pl.BlockSpec((BM, 128), lambda i: (i, 0)),
      out_shape=jax.ShapeDtypeStruct((NPAD, 128), jnp.float32),
  )


def _chunked(a):
  npad, d = a.shape
  return a.reshape(npad, d // 128, 128).transpose(1, 0, 2)


@jax.jit
def kernel(x, edge_index, Wl0, bl0, Wr0, Wl1, bl1, Wr1, Wl2, bl2, Wr2):
  srcp = jnp.concatenate(
      [edge_index[0], jnp.zeros((EPAD - N_EDGES,), jnp.int32)]).reshape(
          NTILE, NB, BEDGE)
  dstp = jnp.concatenate(
      [edge_index[1], jnp.full((EPAD - N_EDGES,), DUMMY, jnp.int32)]).reshape(
          NTILE, NB, BEDGE)
  zeros640 = jnp.zeros((640, 128), jnp.float32)
  ones128 = jnp.ones((BEDGE, 128), jnp.float32)

  xc = _chunked(jnp.pad(x, ((0, NPAD - N_NODES), (0, 0))))

  p0a, p0b, pdeg = _make_sc_agg(2, True)(xc[0], xc[1], srcp, dstp,
                                         zeros640, ones128)
  p0 = jnp.stack([p0a, p0b], axis=0)
  h1 = _make_tc_layer0()(p0, pdeg, xc, Wl0, bl0.reshape(1, -1), Wr0)

  p1s = _make_sc_agg(4, False)(h1[0], h1[1], h1[2], h1[3], srcp, dstp,
                               zeros640)
  p1 = jnp.stack(p1s, axis=0)
  w2 = jnp.concatenate([Wl2, Wr2], axis=1)
  z, r = _make_tc_layer1()(p1, pdeg, h1, Wl1, bl1.reshape(1, -1), Wr1, w2)

  (p2,) = _make_sc_agg(1, False)(z, srcp, dstp, zeros640)
  out = _make_tc_post2()(p2, pdeg, r, bl2.reshape(1, -1))
  return out[:N_NODES]


# speedup vs baseline: 1.4865x; 1.4865x over previous
import functools

import jax
import jax.numpy as jnp
from jax import lax
from jax.experimental import pallas as pl
from jax.experimental.pallas import tpu as pltpu
from jax.experimental.pallas import tpu_sc as plsc

N_NODES = 10000
N_EDGES = 160000
NPAD = 10240
DUMMY = N_NODES
NSC = 2
NTILE = 16
BEDGE = 128
NB0 = 72
NB1 = 8
SB = 24
NBTOT = 96
EPAD = NTILE * NBTOT * BEDGE
ROWS_PER_TILE = NPAD // NTILE
BM = 256


def _make_sc_agg(nchk, with_deg):
  mesh = plsc.VectorSubcoreMesh(core_axis_name="c", subcore_axis_name="s")

  out_type = tuple(
      jax.ShapeDtypeStruct((NSC, NPAD, 128), jnp.float32)
      for _ in range(nchk + (1 if with_deg else 0)))
  scratch = [
      pltpu.VMEM((SB, BEDGE), jnp.int32),
      pltpu.VMEM((SB, BEDGE), jnp.int32),
      pltpu.VMEM((BEDGE, 128), jnp.float32),
      pltpu.VMEM((BEDGE, 128), jnp.float32),
      pltpu.VMEM_SHARED((NPAD, 128), jnp.float32),
      pltpu.SemaphoreType.DMA,
      pltpu.SemaphoreType.DMA,
  ]

  @functools.partial(pl.kernel, mesh=mesh, out_type=out_type,
                     scratch_types=scratch)
  def k(*refs):
    vals = refs[:nchk]
    pos = nchk
    srcp, dstp, zeros_h = refs[pos], refs[pos + 1], refs[pos + 2]
    pos += 3
    if with_deg:
      ones_h = refs[pos]
      pos += 1
    outs = refs[pos:pos + nchk]
    pos += nchk
    if with_deg:
      dout = refs[pos]
      pos += 1
    src_v, dst_v, rows0, rows1, acc, sem0, sem1 = refs[pos:]

    c = lax.axis_index("c")
    s = lax.axis_index("s")
    row0 = s * ROWS_PER_TILE
    nsb = jnp.where(c == 0, NB0 // SB, 1)
    nbs = jnp.where(c == 0, SB, NB1)
    cbase = jnp.where(c == 0, 0, NB0)

    def zero_acc():
      for kk in range(ROWS_PER_TILE // 128):
        pltpu.sync_copy(zeros_h, acc.at[pl.ds(row0 + kk * 128, 128)])

    def flush(out):
      pltpu.sync_copy(acc.at[pl.ds(row0, ROWS_PER_TILE)],
                      out.at[c, pl.ds(row0, ROWS_PER_TILE)])

    def load_window(t):
      off = cbase + t * SB
      pltpu.sync_copy(srcp.at[s, pl.ds(off, SB)], src_v)
      pltpu.sync_copy(dstp.at[s, pl.ds(off, SB)], dst_v)

    for ck in range(nchk):
      vck = vals[ck]
      zero_acc()
      plsc.subcore_barrier()

      def window(t, carry):
        load_window(t)
        pltpu.async_copy(vck.at[src_v.at[0]], rows0, sem0)

        def batch2(i, carry2):
          j = 2 * i
          pltpu.make_async_copy(vck.at[src_v.at[j]], rows0, sem0).wait()
          pltpu.async_copy(vck.at[src_v.at[j + 1]], rows1, sem1)
          pltpu.sync_copy(rows0, acc.at[dst_v.at[j]], add=True)
          pltpu.make_async_copy(vck.at[src_v.at[j + 1]], rows1, sem1).wait()
          pltpu.async_copy(vck.at[src_v.at[j + 2]], rows0, sem0)
          pltpu.sync_copy(rows1, acc.at[dst_v.at[j + 1]], add=True)
          return carry2

        lax.fori_loop(0, nbs // 2 - 1, batch2, 0)
        je = nbs - 2
        pltpu.make_async_copy(vck.at[src_v.at[je]], rows0, sem0).wait()
        pltpu.async_copy(vck.at[src_v.at[je + 1]], rows1, sem1)
        pltpu.sync_copy(rows0, acc.at[dst_v.at[je]], add=True)
        pltpu.make_async_copy(vck.at[src_v.at[je + 1]], rows1, sem1).wait()
        pltpu.sync_copy(rows1, acc.at[dst_v.at[je + 1]], add=True)
        return carry

      lax.fori_loop(0, nsb, window, 0)
      plsc.subcore_barrier()

      flush(outs[ck])
      plsc.subcore_barrier()

    if with_deg:
      pltpu.sync_copy(ones_h, rows0)
      zero_acc()
      plsc.subcore_barrier()

      def dwindow(t, carry):
        load_window(t)

        def dbatch(j, carry2):
          pltpu.sync_copy(rows0, acc.at[dst_v.at[j]], add=True)
          return carry2

        lax.fori_loop(0, nbs, dbatch, 0)
        return carry

      lax.fori_loop(0, nsb, dwindow, 0)
      plsc.subcore_barrier()
      flush(dout)

  return k


def _deg_inv(pd_ref):
  deg = pd_ref[0, :, 0:1] + pd_ref[1, :, 0:1]
  return 1.0 / jnp.maximum(deg, 1.0)


def _make_tc_layer0():
  grid = (NPAD // BM,)

  def body(p_ref, pd_ref, x_ref, wl_ref, bl_ref, wr_ref, o_ref):
    inv = _deg_inv(pd_ref)
    x = jnp.concatenate([x_ref[cc] for cc in range(2)], axis=-1)
    acc = jnp.dot(x, wr_ref[...], preferred_element_type=jnp.float32)
    acc += bl_ref[...]
    agg = jnp.concatenate(
        [p_ref[0, cc] + p_ref[1, cc] for cc in range(2)], axis=-1) * inv
    acc += jnp.dot(agg, wl_ref[...], preferred_element_type=jnp.float32)
    h = jnp.maximum(acc, 0.0)
    for co in range(4):
      o_ref[co] = h[:, co * 128:(co + 1) * 128]

  return pl.pallas_call(
      body,
      grid=grid,
      in_specs=[
          pl.BlockSpec((NSC, 2, BM, 128), lambda i: (0, 0, i, 0)),
          pl.BlockSpec((NSC, BM, 128), lambda i: (0, i, 0)),
          pl.BlockSpec((2, BM, 128), lambda i: (0, i, 0)),
          pl.BlockSpec((256, 512), lambda i: (0, 0)),
          pl.BlockSpec((1, 512), lambda i: (0, 0)),
          pl.BlockSpec((256, 512), lambda i: (0, 0)),
      ],
      out_specs=pl.BlockSpec((4, BM, 128), lambda i: (0, i, 0)),
      out_shape=jax.ShapeDtypeStruct((4, NPAD, 128), jnp.float32),
  )


def _make_tc_layer1():
  grid = (NPAD // BM,)

  def body(p_ref, pd_ref, x_ref, wl_ref, bl_ref, wr_ref, w2_ref,
           z_ref, r_ref):
    inv = _deg_inv(pd_ref)
    x = jnp.concatenate([x_ref[cc] for cc in range(4)], axis=-1)
    acc = jnp.dot(x, wr_ref[...], preferred_element_type=jnp.float32)
    acc += bl_ref[...]
    agg = jnp.concatenate(
        [p_ref[0, cc] + p_ref[1, cc] for cc in range(4)], axis=-1) * inv
    acc += jnp.dot(agg, wl_ref[...], preferred_element_type=jnp.float32)
    h = jnp.maximum(acc, 0.0)
    zr = jnp.dot(h, w2_ref[...], preferred_element_type=jnp.float32)
    z_ref[...] = zr[:, :128]
    r_ref[...] = zr[:, 128:]

  return pl.pallas_call(
      body,
      grid=grid,
      in_specs=[
          pl.BlockSpec((NSC, 4, BM, 128), lambda i: (0, 0, i, 0)),
          pl.BlockSpec((NSC, BM, 128), lambda i: (0, i, 0)),
          pl.BlockSpec((4, BM, 128), lambda i: (0, i, 0)),
          pl.BlockSpec((512, 512), lambda i: (0, 0)),
          pl.BlockSpec((1, 512), lambda i: (0, 0)),
          pl.BlockSpec((512, 512), lambda i: (0, 0)),
          pl.BlockSpec((512, 256), lambda i: (0, 0)),
      ],
      out_specs=[
          pl.BlockSpec((BM, 128), lambda i: (i, 0)),
          pl.BlockSpec((BM, 128), lambda i: (i, 0)),
      ],
      out_shape=[
          jax.ShapeDtypeStruct((NPAD, 128), jnp.float32),
          jax.ShapeDtypeStruct((NPAD, 128), jnp.float32),
      ],
  )


def _make_tc_post2():
  grid = (NPAD // BM,)

  def body(p_ref, pd_ref, r_ref, bl_ref, o_ref):
    inv = _deg_inv(pd_ref)
    o_ref[...] = (p_ref[0] + p_ref[1]) * inv + r_ref[...] + bl_ref[...]

  return pl.pallas_call(
      body,
      grid=grid,
      in_specs=[
          pl.BlockSpec((NSC, BM, 128), lambda i: (0, i, 0)),
          pl.BlockSpec((NSC, BM, 128), lambda i: (0, i, 0)),
          pl.BlockSpec((BM, 128), lambda i: (i, 0)),
          pl.BlockSpec((1, 128), lambda i: (0, 0)),
      ],
      out_specs=pl.BlockSpec((BM, 128), lambda i: (i, 0)),
      out_shape=jax.ShapeDtypeStruct((NPAD, 128), jnp.float32),
  )


def _chunked(a):
  npad, d = a.shape
  return a.reshape(npad, d // 128, 128).transpose(1, 0, 2)


def _edge_layout(e, fill):
  flat = jnp.concatenate([e, jnp.full((EPAD - N_EDGES,), fill, jnp.int32)])
  n0 = NTILE * NB0 * BEDGE
  n1 = NTILE * NB1 * BEDGE
  e0 = flat[:n0].reshape(NTILE, NB0, BEDGE)
  e1 = flat[n0:n0 + n1].reshape(NTILE, NB1, BEDGE)
  ep = flat[n0 + n1:].reshape(NTILE, NBTOT - NB0 - NB1, BEDGE)
  return jnp.concatenate([e0, e1, ep], axis=1)


@jax.jit
def kernel(x, edge_index, Wl0, bl0, Wr0, Wl1, bl1, Wr1, Wl2, bl2, Wr2):
  srcp = _edge_layout(edge_index[0], 0)
  dstp = _edge_layout(edge_index[1], DUMMY)
  zeros128 = jnp.zeros((128, 128), jnp.float32)
  ones128 = jnp.ones((128, 128), jnp.float32)

  xc = _chunked(jnp.pad(x, ((0, NPAD - N_NODES), (0, 0))))

  p0a, p0b, pdeg = _make_sc_agg(2, True)(xc[0], xc[1], srcp, dstp,
                                         zeros128, ones128)
  p0 = jnp.stack([p0a, p0b], axis=1)
  h1 = _make_tc_layer0()(p0, pdeg, xc, Wl0, bl0.reshape(1, -1), Wr0)

  p1s = _make_sc_agg(4, False)(h1[0], h1[1], h1[2], h1[3], srcp, dstp,
                               zeros128)
  p1 = jnp.stack(p1s, axis=1)
  w2 = jnp.concatenate([Wl2, Wr2], axis=1)
  z, r = _make_tc_layer1()(p1, pdeg, h1, Wl1, bl1.reshape(1, -1), Wr1, w2)

  (p2,) = _make_sc_agg(1, False)(z, srcp, dstp, zeros128)
  out = _make_tc_post2()(p2, pdeg, r, bl2.reshape(1, -1))
  return out[:N_NODES]


# speedup vs baseline: 1.5706x; 1.0566x over previous
import functools

import jax
import jax.numpy as jnp
from jax import lax
from jax.experimental import pallas as pl
from jax.experimental.pallas import tpu as pltpu
from jax.experimental.pallas import tpu_sc as plsc

N_NODES = 10000
N_EDGES = 160000
NPAD = 10240
DUMMY = N_NODES
NSC = 2
NTILE = 16
BEDGE = 128
NB0 = 72
NB1 = 8
SB = 24
NBTOT = 96
EPAD = NTILE * NBTOT * BEDGE
ROWS_PER_TILE = NPAD // NTILE
BM = 256


def _make_sc_agg(nchk, with_deg):
  mesh = plsc.VectorSubcoreMesh(core_axis_name="c", subcore_axis_name="s")

  sums_t = jax.ShapeDtypeStruct((nchk, NSC, NPAD, 128), jnp.float32)
  deg_t = jax.ShapeDtypeStruct((NSC, NPAD, 128), jnp.float32)
  out_type = (sums_t, deg_t) if with_deg else sums_t
  scratch = [
      pltpu.VMEM((SB, BEDGE), jnp.int32),
      pltpu.VMEM((SB, BEDGE), jnp.int32),
      pltpu.VMEM((BEDGE, 128), jnp.float32),
      pltpu.VMEM((BEDGE, 128), jnp.float32),
      pltpu.VMEM_SHARED((NPAD, 128), jnp.float32),
      pltpu.SemaphoreType.DMA,
      pltpu.SemaphoreType.DMA,
  ]

  @functools.partial(pl.kernel, mesh=mesh, out_type=out_type,
                     scratch_types=scratch)
  def k(*refs):
    vals_r = refs[0]
    pos = 1
    srcp, dstp, zeros_h = refs[pos], refs[pos + 1], refs[pos + 2]
    pos += 3
    if with_deg:
      ones_h = refs[pos]
      pos += 1
    outs_r = refs[pos]
    pos += 1
    if with_deg:
      dout = refs[pos]
      pos += 1
    src_v, dst_v, rows0, rows1, acc, sem0, sem1 = refs[pos:]

    c = lax.axis_index("c")
    s = lax.axis_index("s")
    row0 = s * ROWS_PER_TILE
    nsb = jnp.where(c == 0, NB0 // SB, 1)
    nbs = jnp.where(c == 0, SB, NB1)
    cbase = jnp.where(c == 0, 0, NB0)

    def zero_acc():
      for kk in range(ROWS_PER_TILE // 128):
        pltpu.sync_copy(zeros_h, acc.at[pl.ds(row0 + kk * 128, 128)])

    def flush(out):
      pltpu.sync_copy(acc.at[pl.ds(row0, ROWS_PER_TILE)],
                      out.at[c, pl.ds(row0, ROWS_PER_TILE)])

    def load_window(t):
      off = cbase + t * SB
      pltpu.sync_copy(srcp.at[s, pl.ds(off, SB)], src_v)
      pltpu.sync_copy(dstp.at[s, pl.ds(off, SB)], dst_v)

    for ck in range(nchk):
      vck = vals_r.at[ck] if nchk > 1 else vals_r
      zero_acc()
      plsc.subcore_barrier()

      def window(t, carry):
        load_window(t)
        pltpu.async_copy(vck.at[src_v.at[0]], rows0, sem0)

        def batch2(i, carry2):
          j = 2 * i
          pltpu.make_async_copy(vck.at[src_v.at[j]], rows0, sem0).wait()
          pltpu.async_copy(vck.at[src_v.at[j + 1]], rows1, sem1)
          pltpu.sync_copy(rows0, acc.at[dst_v.at[j]], add=True)
          pltpu.make_async_copy(vck.at[src_v.at[j + 1]], rows1, sem1).wait()
          pltpu.async_copy(vck.at[src_v.at[j + 2]], rows0, sem0)
          pltpu.sync_copy(rows1, acc.at[dst_v.at[j + 1]], add=True)
          return carry2

        lax.fori_loop(0, nbs // 2 - 1, batch2, 0)
        je = nbs - 2
        pltpu.make_async_copy(vck.at[src_v.at[je]], rows0, sem0).wait()
        pltpu.async_copy(vck.at[src_v.at[je + 1]], rows1, sem1)
        pltpu.sync_copy(rows0, acc.at[dst_v.at[je]], add=True)
        pltpu.make_async_copy(vck.at[src_v.at[je + 1]], rows1, sem1).wait()
        pltpu.sync_copy(rows1, acc.at[dst_v.at[je + 1]], add=True)
        return carry

      lax.fori_loop(0, nsb, window, 0)
      plsc.subcore_barrier()

      flush(outs_r.at[ck])
      plsc.subcore_barrier()

    if with_deg:
      pltpu.sync_copy(ones_h, rows0)
      zero_acc()
      plsc.subcore_barrier()

      def dwindow(t, carry):
        load_window(t)

        def dbatch(j, carry2):
          pltpu.sync_copy(rows0, acc.at[dst_v.at[j]], add=True)
          return carry2

        lax.fori_loop(0, nbs, dbatch, 0)
        return carry

      lax.fori_loop(0, nsb, dwindow, 0)
      plsc.subcore_barrier()
      flush(dout)

  return k


def _deg_inv(pd_ref):
  deg = pd_ref[0, :, 0:1] + pd_ref[1, :, 0:1]
  return 1.0 / jnp.maximum(deg, 1.0)


def _make_tc_layer0():
  grid = (NPAD // BM,)

  def body(p_ref, pd_ref, x_ref, wl_ref, bl_ref, wr_ref, o_ref):
    inv = _deg_inv(pd_ref)
    x = jnp.concatenate([x_ref[cc] for cc in range(2)], axis=-1)
    acc = jnp.dot(x, wr_ref[...], preferred_element_type=jnp.float32)
    acc += bl_ref[...]
    agg = jnp.concatenate(
        [p_ref[cc, 0] + p_ref[cc, 1] for cc in range(2)], axis=-1) * inv
    acc += jnp.dot(agg, wl_ref[...], preferred_element_type=jnp.float32)
    h = jnp.maximum(acc, 0.0)
    for co in range(4):
      o_ref[co] = h[:, co * 128:(co + 1) * 128]

  return pl.pallas_call(
      body,
      grid=grid,
      in_specs=[
          pl.BlockSpec((2, NSC, BM, 128), lambda i: (0, 0, i, 0)),
          pl.BlockSpec((NSC, BM, 128), lambda i: (0, i, 0)),
          pl.BlockSpec((2, BM, 128), lambda i: (0, i, 0)),
          pl.BlockSpec((256, 512), lambda i: (0, 0)),
          pl.BlockSpec((1, 512), lambda i: (0, 0)),
          pl.BlockSpec((256, 512), lambda i: (0, 0)),
      ],
      out_specs=pl.BlockSpec((4, BM, 128), lambda i: (0, i, 0)),
      out_shape=jax.ShapeDtypeStruct((4, NPAD, 128), jnp.float32),
  )


def _make_tc_layer1():
  grid = (NPAD // BM,)

  def body(p_ref, pd_ref, x_ref, wl_ref, bl_ref, wr_ref, w2_ref,
           z_ref, r_ref):
    inv = _deg_inv(pd_ref)
    x = jnp.concatenate([x_ref[cc] for cc in range(4)], axis=-1)
    acc = jnp.dot(x, wr_ref[...], preferred_element_type=jnp.float32)
    acc += bl_ref[...]
    agg = jnp.concatenate(
        [p_ref[cc, 0] + p_ref[cc, 1] for cc in range(4)], axis=-1) * inv
    acc += jnp.dot(agg, wl_ref[...], preferred_element_type=jnp.float32)
    h = jnp.maximum(acc, 0.0)
    zr = jnp.dot(h, w2_ref[...], preferred_element_type=jnp.float32)
    z_ref[...] = zr[:, :128]
    r_ref[...] = zr[:, 128:]

  return pl.pallas_call(
      body,
      grid=grid,
      in_specs=[
          pl.BlockSpec((4, NSC, BM, 128), lambda i: (0, 0, i, 0)),
          pl.BlockSpec((NSC, BM, 128), lambda i: (0, i, 0)),
          pl.BlockSpec((4, BM, 128), lambda i: (0, i, 0)),
          pl.BlockSpec((512, 512), lambda i: (0, 0)),
          pl.BlockSpec((1, 512), lambda i: (0, 0)),
          pl.BlockSpec((512, 512), lambda i: (0, 0)),
          pl.BlockSpec((512, 256), lambda i: (0, 0)),
      ],
      out_specs=[
          pl.BlockSpec((BM, 128), lambda i: (i, 0)),
          pl.BlockSpec((BM, 128), lambda i: (i, 0)),
      ],
      out_shape=[
          jax.ShapeDtypeStruct((NPAD, 128), jnp.float32),
          jax.ShapeDtypeStruct((NPAD, 128), jnp.float32),
      ],
  )


def _make_tc_post2():
  grid = (NPAD // BM,)

  def body(p_ref, pd_ref, r_ref, bl_ref, o_ref):
    inv = _deg_inv(pd_ref)
    o_ref[...] = (p_ref[0, 0] + p_ref[0, 1]) * inv + r_ref[...] + bl_ref[...]

  return pl.pallas_call(
      body,
      grid=grid,
      in_specs=[
          pl.BlockSpec((1, NSC, BM, 128), lambda i: (0, 0, i, 0)),
          pl.BlockSpec((NSC, BM, 128), lambda i: (0, i, 0)),
          pl.BlockSpec((BM, 128), lambda i: (i, 0)),
          pl.BlockSpec((1, 128), lambda i: (0, 0)),
      ],
      out_specs=pl.BlockSpec((BM, 128), lambda i: (i, 0)),
      out_shape=jax.ShapeDtypeStruct((NPAD, 128), jnp.float32),
  )


def _chunked(a):
  npad, d = a.shape
  return a.reshape(npad, d // 128, 128).transpose(1, 0, 2)


def _edge_layout(e, fill):
  flat = jnp.concatenate([e, jnp.full((EPAD - N_EDGES,), fill, jnp.int32)])
  n0 = NTILE * NB0 * BEDGE
  n1 = NTILE * NB1 * BEDGE
  e0 = flat[:n0].reshape(NTILE, NB0, BEDGE)
  e1 = flat[n0:n0 + n1].reshape(NTILE, NB1, BEDGE)
  ep = flat[n0 + n1:].reshape(NTILE, NBTOT - NB0 - NB1, BEDGE)
  return jnp.concatenate([e0, e1, ep], axis=1)


@jax.jit
def kernel(x, edge_index, Wl0, bl0, Wr0, Wl1, bl1, Wr1, Wl2, bl2, Wr2):
  srcp = _edge_layout(edge_index[0], 0)
  dstp = _edge_layout(edge_index[1], DUMMY)
  zeros128 = jnp.zeros((128, 128), jnp.float32)
  ones128 = jnp.ones((128, 128), jnp.float32)

  xc = _chunked(jnp.pad(x, ((0, NPAD - N_NODES), (0, 0))))

  p0, pdeg = _make_sc_agg(2, True)(xc, srcp, dstp, zeros128, ones128)
  h1 = _make_tc_layer0()(p0, pdeg, xc, Wl0, bl0.reshape(1, -1), Wr0)

  p1 = _make_sc_agg(4, False)(h1, srcp, dstp, zeros128)
  w2 = jnp.concatenate([Wl2, Wr2], axis=1)
  z, r = _make_tc_layer1()(p1, pdeg, h1, Wl1, bl1.reshape(1, -1), Wr1, w2)

  p2 = _make_sc_agg(1, False)(z, srcp, dstp, zeros128)
  out = _make_tc_post2()(p2, pdeg, r, bl2.reshape(1, -1))
  return out[:N_NODES]
